# Initial kernel scaffold; baseline (speedup 1.0000x reference)
import jax
import jax.numpy as jnp
from jax.experimental import pallas as pl


def kernel(x, edge_index, Wl1, Wr1, att1, b1, Wl2, Wr2, att2, b2, Wl3, Wr3, att3, b3, g1, be1, rm1, rv1, g2, be2, rm2, rv2, g3, be3, rm3, rv3, Wm1, bm1, Wm2, bm2, Wm3, bm3):
    raise NotImplementedError("write your pallas kernel here")


# speedup vs baseline: 10.3716x; 10.3716x over previous
import functools

import jax
import jax.numpy as jnp
from jax import lax
from jax.experimental import pallas as pl
from jax.experimental.pallas import tpu as pltpu
from jax.experimental.pallas import tpu_sc as plsc

H = 4
K_EDGES = 32
NUM_TILES = 32


def _largest_div(n, cap):
    for d in range(min(n, cap), 0, -1):
        if n % d == 0:
            return d
    return 1


def _gat_edge_sc(xl, xr, src, dstg, dsc, att, n, c):
    cp = c + 16
    epad = src.shape[0]
    assert epad % (NUM_TILES * K_EDGES) == 0
    epw = epad // NUM_TILES
    nchunk = epw // K_EDGES
    npt = -(-(-(-n // 16)) // 128) * 128
    while 16 * npt <= n:
        npt += 128
    n_pad = 16 * npt
    zr = 128
    nz = npt // zr
    cblk = c // 16

    mesh = plsc.VectorSubcoreMesh(core_axis_name="c", subcore_axis_name="s",
                                  num_cores=2, num_subcores=16)

    @functools.partial(
        pl.kernel,
        out_type=jax.ShapeDtypeStruct((2, H, n_pad, cp), jnp.float32),
        mesh=mesh,
        scratch_types=[
            pltpu.VMEM((K_EDGES,), jnp.int32),
            pltpu.VMEM((K_EDGES,), jnp.int32),
            pltpu.VMEM((K_EDGES,), jnp.int32),
            pltpu.VMEM((K_EDGES,), jnp.int32),
            pltpu.VMEM((K_EDGES,), jnp.int32),
            pltpu.VMEM((K_EDGES, c), jnp.float32),
            pltpu.VMEM((K_EDGES, c), jnp.float32),
            pltpu.VMEM((K_EDGES, cp), jnp.float32),
            pltpu.VMEM((c,), jnp.float32),
            pltpu.VMEM((zr, cp), jnp.float32),
            pltpu.VMEM_SHARED((n_pad, cp), jnp.float32),
        ],
        compiler_params=pltpu.CompilerParams(needs_layout_passes=False,
                                             use_tc_tiling_on_sc=False),
    )
    def k(xl_hbm, xr_hbm, src_hbm, dstg_hbm, dsc_hbm, att_hbm, out_hbm,
          srcv, dgv, dsv, srchv, dsthv, rl, rr, buf, attv, zbuf, acc):
        core = lax.axis_index("c")
        sub = lax.axis_index("s")
        ebase = core * (epad // 2) + sub * epw
        row0 = sub * npt

        z16 = jnp.zeros((16,), jnp.float32)

        @pl.loop(0, zr)
        def _(i):
            for cb in range(cp // 16):
                zbuf[i, pl.ds(cb * 16, 16)] = z16

        @pl.loop(0, H)
        def _head(h):
            for j in range(nz):
                pltpu.sync_copy(zbuf, acc.at[pl.ds(row0 + j * zr, zr)])
            pltpu.sync_copy(att_hbm.at[h], attv)
            att_b = [attv[pl.ds(cb * 16, 16)] for cb in range(cblk)]
            hn = h * n
            plsc.subcore_barrier()

            @pl.loop(0, nchunk)
            def _chunk(g):
                e0 = ebase + g * K_EDGES
                pltpu.sync_copy(src_hbm.at[pl.ds(e0, K_EDGES)], srcv)
                pltpu.sync_copy(dstg_hbm.at[pl.ds(e0, K_EDGES)], dgv)
                pltpu.sync_copy(dsc_hbm.at[pl.ds(e0, K_EDGES)], dsv)
                for j in range(K_EDGES // 16):
                    sl = pl.ds(j * 16, 16)
                    srchv[sl] = srcv[sl] + hn
                    dsthv[sl] = dgv[sl] + hn
                pltpu.sync_copy(xl_hbm.at[srchv], rl)
                pltpu.sync_copy(xr_hbm.at[dsthv], rr)
                for i in range(K_EDGES):
                    accv = None
                    lblk = []
                    for cb in range(cblk):
                        sl = pl.ds(cb * 16, 16)
                        l = rl[i, sl]
                        r = rr[i, sl]
                        s = l + r
                        lrel = jnp.maximum(s, 0.2 * s)
                        t = lrel * att_b[cb]
                        accv = t if accv is None else accv + t
                        lblk.append(l)
                    ex = jnp.exp(jnp.full((16,), jnp.sum(accv)))
                    for cb in range(cblk):
                        buf[i, pl.ds(cb * 16, 16)] = lblk[cb] * ex
                    lane = lax.iota(jnp.int32, 16)
                    buf[i, pl.ds(c, 16)] = jnp.where(lane == 0, ex, 0.0)
                pltpu.sync_copy(buf, acc.at[dsv], add=True)

            plsc.subcore_barrier()
            pltpu.sync_copy(acc.at[pl.ds(row0, npt)],
                            out_hbm.at[core, h, pl.ds(row0, npt)])
            plsc.subcore_barrier()

    return k(xl, xr, src, dstg, dsc, att)


def _proj_heads(xs, wlh, wrh, c):
    n = xs[0].shape[0]
    nblk = _largest_div(n, 1024)
    f_tot = wlh.shape[1]
    splits = [x.shape[1] for x in xs]

    def body(*refs):
        xrefs = refs[:len(xs)]
        wl_ref, wr_ref, xl_ref, xr_ref = refs[len(xs):]
        for h in range(H):
            accl = None
            accr = None
            off = 0
            for xi, fi in zip(xrefs, splits):
                xb = xi[...]
                pl_w = wl_ref[h, pl.ds(off, fi), :]
                pr_w = wr_ref[h, pl.ds(off, fi), :]
                dl = jnp.dot(xb, pl_w, preferred_element_type=jnp.float32)
                dr = jnp.dot(xb, pr_w, preferred_element_type=jnp.float32)
                accl = dl if accl is None else accl + dl
                accr = dr if accr is None else accr + dr
                off += fi
            xl_ref[h] = accl
            xr_ref[h] = accr

    grid = (n // nblk,)
    in_specs = [pl.BlockSpec((nblk, fi), lambda i: (i, 0)) for fi in splits]
    in_specs += [pl.BlockSpec((H, f_tot, c), lambda i: (0, 0, 0))] * 2
    out_specs = [pl.BlockSpec((H, nblk, c), lambda i: (0, i, 0))] * 2
    out_shape = [jax.ShapeDtypeStruct((H, n, c), jnp.float32)] * 2
    xl3, xr3 = pl.pallas_call(
        body, grid=grid, in_specs=in_specs, out_specs=out_specs,
        out_shape=out_shape)(*xs, wlh, wrh)
    return xl3.reshape(H * n, c), xr3.reshape(H * n, c)


def _combine(acc, b, g, be, rm, rv, n, c):
    cp = c + 16
    nblk = _largest_div(n, 1024)

    def body(acc_ref, b_ref, g_ref, be_ref, rm_ref, rv_ref, o_ref):
        for h in range(H):
            a = acc_ref[0, h] + acc_ref[1, h]
            num = a[:, :c]
            den = a[:, c:c + 1]
            v = num / (den + 1e-16) + b_ref[h]
            v = jnp.maximum(v, 0.0)
            v = (v - rm_ref[h]) * jax.lax.rsqrt(rv_ref[h] + 1e-5)
            o_ref[:, h, :] = v * g_ref[h] + be_ref[h]

    grid = (n // nblk,)
    vec = pl.BlockSpec((H, 1, c), lambda i: (0, 0, 0))
    r3 = lambda a: a.reshape(H, 1, c)
    return pl.pallas_call(
        body, grid=grid,
        in_specs=[pl.BlockSpec((2, H, nblk, cp), lambda i: (0, 0, i, 0)),
                  vec, vec, vec, vec, vec],
        out_specs=pl.BlockSpec((nblk, H, c), lambda i: (i, 0, 0)),
        out_shape=jax.ShapeDtypeStruct((n, H, c), jnp.float32),
    )(acc, r3(b), r3(g), r3(be), r3(rm), r3(rv))


def _colmean(x):
    n, f = x.shape
    nblk = _largest_div(n, 1024)

    def body(x_ref, o_ref):
        @pl.when(pl.program_id(0) == 0)
        def _():
            o_ref[...] = jnp.zeros_like(o_ref)
        o_ref[...] += jnp.sum(x_ref[...], axis=0, keepdims=True) * (1.0 / n)

    return pl.pallas_call(
        body, grid=(n // nblk,),
        in_specs=[pl.BlockSpec((nblk, f), lambda i: (i, 0))],
        out_specs=pl.BlockSpec((1, f), lambda i: (0, 0)),
        out_shape=jax.ShapeDtypeStruct((1, f), jnp.float32))(x)


def _mlp(mx, m1, m2, m3, w1, b1, w2, b2, w3, b3):
    f0, f1, f2, f3 = mx.shape[1], m1.shape[1], m2.shape[1], m3.shape[1]

    def body(mx_r, m1_r, m2_r, m3_r, w1_r, b1_r, w2_r, b2_r, w3_r, b3_r, o):
        h = (jnp.dot(mx_r[...], w1_r[pl.ds(0, f0), :],
                     preferred_element_type=jnp.float32)
             + jnp.dot(m1_r[...], w1_r[pl.ds(f0, f1), :],
                       preferred_element_type=jnp.float32)
             + jnp.dot(m2_r[...], w1_r[pl.ds(f0 + f1, f2), :],
                       preferred_element_type=jnp.float32)
             + jnp.dot(m3_r[...], w1_r[pl.ds(f0 + f1 + f2, f3), :],
                       preferred_element_type=jnp.float32)
             + b1_r[...])
        h = jnp.maximum(h, 0.0)
        h2 = jnp.maximum(
            jnp.dot(h, w2_r[...], preferred_element_type=jnp.float32)
            + b2_r[...], 0.0)
        o[...] = (jnp.dot(h2, w3_r[...], preferred_element_type=jnp.float32)
                  + b3_r[...])

    nout = b3.shape[0]
    return pl.pallas_call(
        body,
        out_shape=jax.ShapeDtypeStruct((1, nout), jnp.float32),
    )(mx, m1, m2, m3, w1, b1.reshape(1, -1), w2, b2.reshape(1, -1),
      w3, b3.reshape(1, -1))


def _gat_layer(xs, wl, wr, att, b, g, be, rm, rv, src, dstg, dsc, n, c):
    f_tot = wl.shape[0]
    wlh = wl.reshape(f_tot, H, c).transpose(1, 0, 2)
    wrh = wr.reshape(f_tot, H, c).transpose(1, 0, 2)
    xl, xr = _proj_heads(xs, wlh, wrh, c)
    acc = _gat_edge_sc(xl, xr, src, dstg, dsc, att, n, c)
    return _combine(acc, b, g, be, rm, rv, n, c).reshape(n, H * c)


def kernel(x, edge_index, Wl1, Wr1, att1, b1, Wl2, Wr2, att2, b2,
           Wl3, Wr3, att3, b3, g1, be1, rm1, rv1, g2, be2, rm2, rv2,
           g3, be3, rm3, rv3, Wm1, bm1, Wm2, bm2, Wm3, bm3):
    n = x.shape[0]
    e = edge_index.shape[1]
    loop = jnp.arange(n, dtype=jnp.int32)
    src = jnp.concatenate([edge_index[0].astype(jnp.int32), loop])
    dst = jnp.concatenate([edge_index[1].astype(jnp.int32), loop])
    etot = e + n
    epad = -(-etot // (NUM_TILES * K_EDGES)) * (NUM_TILES * K_EDGES)
    pad = epad - etot
    src_p = jnp.pad(src, (0, pad))
    dstg_p = jnp.pad(dst, (0, pad))
    dsc_p = jnp.pad(dst, (0, pad), constant_values=n)

    x1 = _gat_layer([x], Wl1, Wr1, att1, b1, g1, be1, rm1, rv1,
                    src_p, dstg_p, dsc_p, n, 128)
    x2 = _gat_layer([x1], Wl2, Wr2, att2, b2, g2, be2, rm2, rv2,
                    src_p, dstg_p, dsc_p, n, 64)
    x3 = _gat_layer([x, x2], Wl3, Wr3, att3, b3, g3, be3, rm3, rv3,
                    src_p, dstg_p, dsc_p, n, 32)

    return _mlp(_colmean(x), _colmean(x1), _colmean(x2), _colmean(x3),
                Wm1, bm1, Wm2, bm2, Wm3, bm3)


# speedup vs baseline: 26.9910x; 2.6024x over previous
import functools

import jax
import jax.numpy as jnp
from jax import lax
from jax.experimental import pallas as pl
from jax.experimental.pallas import tpu as pltpu
from jax.experimental.pallas import tpu_sc as plsc

H = 4
K_EDGES = 32
NUM_TILES = 32


def _largest_div(n, cap):
    for d in range(min(n, cap), 0, -1):
        if n % d == 0:
            return d
    return 1


def _gat_edge_sc(xl, xr, idx, att, n, c):
    cp = c + 16
    nchunk = idx.shape[1]
    assert idx.shape == (NUM_TILES, nchunk, 3 * K_EDGES) and nchunk % 2 == 0
    npt = -(-(-(-n // 16)) // 128) * 128
    while 16 * npt <= n:
        npt += 128
    n_pad = 16 * npt
    zr = K_EDGES
    assert npt % zr == 0
    nz = npt // zr
    cblk = c // 16

    mesh = plsc.VectorSubcoreMesh(core_axis_name="c", subcore_axis_name="s",
                                  num_cores=2, num_subcores=16)

    @functools.partial(
        pl.kernel,
        out_type=jax.ShapeDtypeStruct((2, H, n_pad, cp), jnp.float32),
        mesh=mesh,
        scratch_types=[
            pltpu.VMEM((nchunk, 3 * K_EDGES), jnp.int32),
            [pltpu.VMEM((K_EDGES,), jnp.int32)] * 2,
            [pltpu.VMEM((K_EDGES,), jnp.int32)] * 2,
            [pltpu.VMEM((K_EDGES,), jnp.int32)] * 2,
            [pltpu.VMEM((K_EDGES, c), jnp.float32)] * 2,
            [pltpu.VMEM((K_EDGES, c), jnp.float32)] * 2,
            [pltpu.VMEM((K_EDGES, cp), jnp.float32)] * 2,
            pltpu.VMEM((c,), jnp.float32),
            pltpu.VMEM_SHARED((n_pad, cp), jnp.float32),
            [pltpu.SemaphoreType.DMA] * 2,
            [pltpu.SemaphoreType.DMA] * 2,
        ],
        compiler_params=pltpu.CompilerParams(needs_layout_passes=False,
                                             use_tc_tiling_on_sc=False),
    )
    def k(xl_hbm, xr_hbm, idx_hbm, att_hbm, out_hbm,
          idxall, srchv, dsthv, dscv, rl, rr, sbuf,
          attv, acc, gsem, ssem):
        core = lax.axis_index("c")
        sub = lax.axis_index("s")
        tid = core * 16 + sub
        row0 = sub * npt

        pltpu.sync_copy(idx_hbm.at[tid], idxall)

        z16 = jnp.zeros((16,), jnp.float32)

        @pl.loop(0, H)
        def _head(h):
            @pl.loop(0, K_EDGES)
            def _(i):
                for cb in range(cp // 16):
                    sbuf[0][i, pl.ds(cb * 16, 16)] = z16

            for j in range(nz):
                pltpu.sync_copy(sbuf[0], acc.at[pl.ds(row0 + j * zr, zr)])
            pltpu.sync_copy(att_hbm.at[h], attv)
            att_b = [attv[pl.ds(cb * 16, 16)] for cb in range(cblk)]
            hn = h * n

            def load_idx(g, b):
                for j in range(K_EDGES // 16):
                    srchv[b][pl.ds(j * 16, 16)] = (
                        idxall[g, pl.ds(j * 16, 16)] + hn)
                    dsthv[b][pl.ds(j * 16, 16)] = (
                        idxall[g, pl.ds(K_EDGES + j * 16, 16)] + hn)

            def start_gather(b):
                pltpu.async_copy(xl_hbm.at[srchv[b]], rl[b], gsem[b])
                pltpu.async_copy(xr_hbm.at[dsthv[b]], rr[b], gsem[b])

            def wait_gather(b):
                pltpu.make_async_copy(xl_hbm.at[srchv[b]], rl[b],
                                      gsem[b]).wait()
                pltpu.make_async_copy(xr_hbm.at[dsthv[b]], rr[b],
                                      gsem[b]).wait()

            def wait_scatter(b):
                pltpu.make_async_copy(sbuf[b], acc.at[dscv[b]],
                                      ssem[b]).wait()

            def compute(g, b):
                for j in range(K_EDGES // 16):
                    dscv[b][pl.ds(j * 16, 16)] = idxall[
                        g, pl.ds(2 * K_EDGES + j * 16, 16)]
                for i in range(K_EDGES):
                    accv = None
                    for cb in range(cblk):
                        sl = pl.ds(cb * 16, 16)
                        s = rl[b][i, sl] + rr[b][i, sl]
                        lrel = jnp.maximum(s, 0.2 * s)
                        t = lrel * att_b[cb]
                        accv = t if accv is None else accv + t
                    ex = jnp.exp(jnp.full((16,), jnp.sum(accv)))
                    for cb in range(cblk):
                        sl = pl.ds(cb * 16, 16)
                        sbuf[b][i, sl] = rl[b][i, sl] * ex
                    lane = lax.iota(jnp.int32, 16)
                    sbuf[b][i, pl.ds(c, 16)] = jnp.where(lane == 0, ex, 0.0)
                pltpu.async_copy(sbuf[b], acc.at[dscv[b]], ssem[b],
                                 add=True)

            plsc.subcore_barrier()

            load_idx(0, 0)
            start_gather(0)

            @pl.loop(0, nchunk, step=2)
            def _chunk(g):
                load_idx(g + 1, 1)
                start_gather(1)
                wait_gather(0)

                @pl.when(g >= 2)
                def _():
                    wait_scatter(0)
                compute(g, 0)

                @pl.when(g + 2 < nchunk)
                def _():
                    load_idx(g + 2, 0)
                    start_gather(0)
                wait_gather(1)

                @pl.when(g >= 2)
                def _():
                    wait_scatter(1)
                compute(g + 1, 1)

            wait_scatter(0)
            wait_scatter(1)
            plsc.subcore_barrier()
            pltpu.sync_copy(acc.at[pl.ds(row0, npt)],
                            out_hbm.at[core, h, pl.ds(row0, npt)])
            plsc.subcore_barrier()

    return k(xl, xr, idx, att)


def _proj_heads(xs, wlh, wrh, c):
    n = xs[0].shape[0]
    nblk = _largest_div(n, 1024)
    f_tot = wlh.shape[1]
    splits = [x.shape[1] for x in xs]

    def body(*refs):
        xrefs = refs[:len(xs)]
        wl_ref, wr_ref, xl_ref, xr_ref = refs[len(xs):]
        for h in range(H):
            accl = None
            accr = None
            off = 0
            for xi, fi in zip(xrefs, splits):
                xb = xi[...]
                pl_w = wl_ref[h, pl.ds(off, fi), :]
                pr_w = wr_ref[h, pl.ds(off, fi), :]
                dl = jnp.dot(xb, pl_w, preferred_element_type=jnp.float32)
                dr = jnp.dot(xb, pr_w, preferred_element_type=jnp.float32)
                accl = dl if accl is None else accl + dl
                accr = dr if accr is None else accr + dr
                off += fi
            xl_ref[h] = accl
            xr_ref[h] = accr

    grid = (n // nblk,)
    in_specs = [pl.BlockSpec((nblk, fi), lambda i: (i, 0)) for fi in splits]
    in_specs += [pl.BlockSpec((H, f_tot, c), lambda i: (0, 0, 0))] * 2
    out_specs = [pl.BlockSpec((H, nblk, c), lambda i: (0, i, 0))] * 2
    out_shape = [jax.ShapeDtypeStruct((H, n, c), jnp.float32)] * 2
    xl3, xr3 = pl.pallas_call(
        body, grid=grid, in_specs=in_specs, out_specs=out_specs,
        out_shape=out_shape)(*xs, wlh, wrh)
    return xl3.reshape(H * n, c), xr3.reshape(H * n, c)


def _combine(acc, b, g, be, rm, rv, n, c):
    cp = c + 16
    nblk = _largest_div(n, 1024)

    def body(acc_ref, b_ref, g_ref, be_ref, rm_ref, rv_ref, o_ref):
        for h in range(H):
            a = acc_ref[0, h] + acc_ref[1, h]
            num = a[:, :c]
            den = a[:, c:c + 1]
            v = num / (den + 1e-16) + b_ref[h]
            v = jnp.maximum(v, 0.0)
            v = (v - rm_ref[h]) * jax.lax.rsqrt(rv_ref[h] + 1e-5)
            o_ref[:, h, :] = v * g_ref[h] + be_ref[h]

    grid = (n // nblk,)
    vec = pl.BlockSpec((H, 1, c), lambda i: (0, 0, 0))
    r3 = lambda a: a.reshape(H, 1, c)
    return pl.pallas_call(
        body, grid=grid,
        in_specs=[pl.BlockSpec((2, H, nblk, cp), lambda i: (0, 0, i, 0)),
                  vec, vec, vec, vec, vec],
        out_specs=pl.BlockSpec((nblk, H, c), lambda i: (i, 0, 0)),
        out_shape=jax.ShapeDtypeStruct((n, H, c), jnp.float32),
    )(acc, r3(b), r3(g), r3(be), r3(rm), r3(rv))


def _colmean(x):
    n, f = x.shape
    nblk = _largest_div(n, 1024)

    def body(x_ref, o_ref):
        @pl.when(pl.program_id(0) == 0)
        def _():
            o_ref[...] = jnp.zeros_like(o_ref)
        o_ref[...] += jnp.sum(x_ref[...], axis=0, keepdims=True) * (1.0 / n)

    return pl.pallas_call(
        body, grid=(n // nblk,),
        in_specs=[pl.BlockSpec((nblk, f), lambda i: (i, 0))],
        out_specs=pl.BlockSpec((1, f), lambda i: (0, 0)),
        out_shape=jax.ShapeDtypeStruct((1, f), jnp.float32))(x)


def _mlp(mx, m1, m2, m3, w1, b1, w2, b2, w3, b3):
    f0, f1, f2, f3 = mx.shape[1], m1.shape[1], m2.shape[1], m3.shape[1]

    def body(mx_r, m1_r, m2_r, m3_r, w1_r, b1_r, w2_r, b2_r, w3_r, b3_r, o):
        h = (jnp.dot(mx_r[...], w1_r[pl.ds(0, f0), :],
                     preferred_element_type=jnp.float32)
             + jnp.dot(m1_r[...], w1_r[pl.ds(f0, f1), :],
                       preferred_element_type=jnp.float32)
             + jnp.dot(m2_r[...], w1_r[pl.ds(f0 + f1, f2), :],
                       preferred_element_type=jnp.float32)
             + jnp.dot(m3_r[...], w1_r[pl.ds(f0 + f1 + f2, f3), :],
                       preferred_element_type=jnp.float32)
             + b1_r[...])
        h = jnp.maximum(h, 0.0)
        h2 = jnp.maximum(
            jnp.dot(h, w2_r[...], preferred_element_type=jnp.float32)
            + b2_r[...], 0.0)
        o[...] = (jnp.dot(h2, w3_r[...], preferred_element_type=jnp.float32)
                  + b3_r[...])

    nout = b3.shape[0]
    return pl.pallas_call(
        body,
        out_shape=jax.ShapeDtypeStruct((1, nout), jnp.float32),
    )(mx, m1, m2, m3, w1, b1.reshape(1, -1), w2, b2.reshape(1, -1),
      w3, b3.reshape(1, -1))


def _gat_layer(xs, wl, wr, att, b, g, be, rm, rv, idx, n, c):
    f_tot = wl.shape[0]
    wlh = wl.reshape(f_tot, H, c).transpose(1, 0, 2)
    wrh = wr.reshape(f_tot, H, c).transpose(1, 0, 2)
    xl, xr = _proj_heads(xs, wlh, wrh, c)
    acc = _gat_edge_sc(xl, xr, idx, att, n, c)
    return _combine(acc, b, g, be, rm, rv, n, c).reshape(n, H * c)


def kernel(x, edge_index, Wl1, Wr1, att1, b1, Wl2, Wr2, att2, b2,
           Wl3, Wr3, att3, b3, g1, be1, rm1, rv1, g2, be2, rm2, rv2,
           g3, be3, rm3, rv3, Wm1, bm1, Wm2, bm2, Wm3, bm3):
    n = x.shape[0]
    e = edge_index.shape[1]
    loop = jnp.arange(n, dtype=jnp.int32)
    src = jnp.concatenate([edge_index[0].astype(jnp.int32), loop])
    dst = jnp.concatenate([edge_index[1].astype(jnp.int32), loop])
    etot = e + n
    quant = NUM_TILES * K_EDGES * 2
    epad = -(-etot // quant) * quant
    pad = epad - etot
    nchunk = epad // (NUM_TILES * K_EDGES)
    shp = (NUM_TILES, nchunk, K_EDGES)
    idx_p = jnp.concatenate(
        [jnp.pad(src, (0, pad)).reshape(shp),
         jnp.pad(dst, (0, pad)).reshape(shp),
         jnp.pad(dst, (0, pad), constant_values=n).reshape(shp)], axis=2)

    x1 = _gat_layer([x], Wl1, Wr1, att1, b1, g1, be1, rm1, rv1,
                    idx_p, n, 128)
    x2 = _gat_layer([x1], Wl2, Wr2, att2, b2, g2, be2, rm2, rv2,
                    idx_p, n, 64)
    x3 = _gat_layer([x, x2], Wl3, Wr3, att3, b3, g3, be3, rm3, rv3,
                    idx_p, n, 32)

    return _mlp(_colmean(x), _colmean(x1), _colmean(x2), _colmean(x3),
                Wm1, bm1, Wm2, bm2, Wm3, bm3)
